# baseline (speedup 1.0000x reference)
import numpy as np
import jax
import jax.numpy as jnp
from jax.experimental import pallas as pl
from jax.experimental.pallas import tpu as pltpu

_PI = float(np.pi)
_SCALE = float(2.0 / (np.cos(0.0) - np.cos(2.0 * np.pi / 8.0)))
_LOG2E = float(np.log2(np.e))


def _poly_coeffs():
    r = np.linspace(-np.pi, np.pi, 40001)
    q = r * r
    V = np.stack([q**i for i in range(6)], axis=1)
    cc = np.linalg.lstsq(V, np.cos(r), rcond=None)[0]
    sc = np.linalg.lstsq(V * r[:, None], np.sin(r), rcond=None)[0]
    return [float(x) for x in cc], [float(x) for x in sc]


_CC, _SC = _poly_coeffs()


def _expand_matrix(nf, nb):
    F = nf * nb
    centers = np.arange(nb) * (2.0 * np.pi / nb)
    A = np.zeros((4 * F, 6 * nf), dtype=np.float64)
    for jj in range(2):
        for k in range(nb):
            for i in range(nf):
                A[jj * F + k * nf + i, jj * nf + i] = -np.cos(centers[k])
                A[jj * F + k * nf + i, 2 * nf + jj * nf + i] = -np.sin(centers[k])
                A[2 * F + jj * F + k * nf + i, 4 * nf + jj * nf + i] = 1.0
    return A.astype(np.float32)


def _body(f_ref, m_ref, a_ref, w4_ref, b4_ref, o_ref):
    bb, D = f_ref.shape[0], o_ref.shape[1]
    F2 = a_ref.shape[0] // 2
    for q0 in range(0, bb, 4):
        feats = []
        for p in (q0, q0 + 2):
            f2 = jnp.concatenate([f_ref[p], f_ref[p + 1]], axis=0)
            m2 = jnp.concatenate([m_ref[p], m_ref[p + 1]], axis=0)
            r = f2 - _PI
            q = r * r
            c = _CC[5]
            s = _SC[5]
            for i in range(4, -1, -1):
                c = c * q + _CC[i]
                s = s * q + _SC[i]
            s = s * r
            csm = jnp.concatenate([c, s, m2], axis=0)
            em = jnp.dot(a_ref[...], csm, preferred_element_type=jnp.float32)
            feats.append(jnp.exp((em[:F2] - 1.0) * _SCALE) * em[F2:])
        feat4 = jnp.concatenate(feats, axis=0)
        out4 = jnp.dot(w4_ref[...], feat4,
                       preferred_element_type=jnp.float32) + b4_ref[...]
        for u in range(4):
            o_ref[q0 + u] = out4[u * D:(u + 1) * D]


def kernel(field, W, b, mask_field):
    B, G, nf = field.shape
    nb = 8
    D = W.shape[1]
    F = nf * nb

    fT = jnp.transpose(field, (0, 2, 1))
    mT = jnp.transpose(mask_field, (0, 2, 1))

    A = _expand_matrix(nf, nb)
    perm = np.asarray([i * nb + k for k in range(nb) for i in range(nf)])
    WT = W.astype(jnp.float32)[perm].T
    W4 = jnp.kron(jnp.eye(4, dtype=jnp.float32), WT)
    b4 = jnp.tile(b.astype(jnp.float32).reshape(D, 1), (4, 1))

    bb = 16 if B % 16 == 0 else 4
    grid = (B // bb,)
    in3 = lambda c2: pl.BlockSpec((bb, c2, G), lambda i: (i, 0, 0))
    const_spec = lambda shape: pl.BlockSpec(shape, lambda i: (0, 0))

    flops = 2 * B * G * (2 * F * 3 * nf + D * F)
    transcendentals = B * G * F
    bytes_accessed = B * G * (2 * nf + D) * 4
    cost = pl.CostEstimate(flops=flops, transcendentals=transcendentals,
                           bytes_accessed=bytes_accessed)

    out = pl.pallas_call(
        _body,
        out_shape=jax.ShapeDtypeStruct((B, D, G), jnp.float32),
        grid_spec=pltpu.PrefetchScalarGridSpec(
            num_scalar_prefetch=0,
            grid=grid,
            in_specs=[in3(nf), in3(nf),
                      const_spec((4 * F, 6 * nf)),
                      const_spec((4 * D, 4 * F)), const_spec((4 * D, 1))],
            out_specs=pl.BlockSpec((bb, D, G), lambda i: (i, 0, 0)),
        ),
        compiler_params=pltpu.CompilerParams(
            dimension_semantics=("parallel",)),
        cost_estimate=cost,
    )(fT, mT, jnp.asarray(A), W4, b4)

    return jnp.transpose(out, (0, 2, 1))


# speedup vs baseline: 1.1547x; 1.1547x over previous
import numpy as np
import jax
import jax.numpy as jnp
from jax.experimental import pallas as pl
from jax.experimental.pallas import tpu as pltpu

_PI = float(np.pi)
_SCALE = float(2.0 / (np.cos(0.0) - np.cos(2.0 * np.pi / 8.0)))
_LOG2E = float(np.log2(np.e))


def _poly_coeffs():
    r = np.linspace(-np.pi, np.pi, 40001)
    q = r * r
    V = np.stack([q**i for i in range(6)], axis=1)
    cc = np.linalg.lstsq(V, np.cos(r), rcond=None)[0]
    sc = np.linalg.lstsq(V * r[:, None], np.sin(r), rcond=None)[0]
    return [float(x) for x in cc], [float(x) for x in sc]


_CC, _SC = _poly_coeffs()


def _expand_matrix(nf, nb):
    F = nf * nb
    centers = np.arange(nb) * (2.0 * np.pi / nb)
    A = np.zeros((4 * F, 6 * nf), dtype=np.float64)
    for jj in range(2):
        for k in range(nb):
            for i in range(nf):
                A[jj * F + k * nf + i, jj * nf + i] = -np.cos(centers[k])
                A[jj * F + k * nf + i, 2 * nf + jj * nf + i] = -np.sin(centers[k])
                A[2 * F + jj * F + k * nf + i, 4 * nf + jj * nf + i] = 1.0
    return A.astype(np.float32)


def _body(f_ref, m_ref, a_ref, w4_ref, b4_ref, o_ref):
    bb, D = f_ref.shape[0], o_ref.shape[1]
    F2 = a_ref.shape[0] // 2
    for q0 in range(0, bb, 4):
        feats = []
        for p in (q0, q0 + 2):
            f2 = jnp.concatenate([f_ref[p], f_ref[p + 1]], axis=0)
            m2 = jnp.concatenate([m_ref[p], m_ref[p + 1]], axis=0)
            r = f2 - _PI
            q = r * r
            c = _CC[5]
            s = _SC[5]
            for i in range(4, -1, -1):
                c = c * q + _CC[i]
                s = s * q + _SC[i]
            s = s * r
            csm = jnp.concatenate([c, s, m2], axis=0)
            em = jnp.dot(a_ref[...], csm, preferred_element_type=jnp.float32)
            feats.append(jnp.exp((em[:F2] - 1.0) * _SCALE) * em[F2:])
        feat4 = jnp.concatenate(feats, axis=0)
        out4 = jnp.dot(w4_ref[...], feat4,
                       preferred_element_type=jnp.float32) + b4_ref[...]
        for u in range(4):
            o_ref[q0 + u] = out4[u * D:(u + 1) * D]


def kernel(field, W, b, mask_field):
    B, G, nf = field.shape
    nb = 8
    D = W.shape[1]
    F = nf * nb

    fT = jnp.transpose(field, (0, 2, 1))
    mT = jnp.transpose(mask_field, (0, 2, 1))

    A = _expand_matrix(nf, nb)
    perm = np.asarray([i * nb + k for k in range(nb) for i in range(nf)])
    WT = W.astype(jnp.float32)[perm].T
    W4 = jnp.kron(jnp.eye(4, dtype=jnp.float32), WT)
    b4 = jnp.tile(b.astype(jnp.float32).reshape(D, 1), (4, 1))

    bb = 64 if B % 64 == 0 else 4
    grid = (B // bb,)
    in3 = lambda c2: pl.BlockSpec((bb, c2, G), lambda i: (i, 0, 0))
    const_spec = lambda shape: pl.BlockSpec(shape, lambda i: (0, 0))

    flops = 2 * B * G * (2 * F * 3 * nf + D * F)
    transcendentals = B * G * F
    bytes_accessed = B * G * (2 * nf + D) * 4
    cost = pl.CostEstimate(flops=flops, transcendentals=transcendentals,
                           bytes_accessed=bytes_accessed)

    out = pl.pallas_call(
        _body,
        out_shape=jax.ShapeDtypeStruct((B, D, G), jnp.float32),
        grid_spec=pltpu.PrefetchScalarGridSpec(
            num_scalar_prefetch=0,
            grid=grid,
            in_specs=[in3(nf), in3(nf),
                      const_spec((4 * F, 6 * nf)),
                      const_spec((4 * D, 4 * F)), const_spec((4 * D, 1))],
            out_specs=pl.BlockSpec((bb, D, G), lambda i: (i, 0, 0)),
        ),
        compiler_params=pltpu.CompilerParams(
            dimension_semantics=("parallel",)),
        cost_estimate=cost,
    )(fT, mT, jnp.asarray(A), W4, b4)

    return jnp.transpose(out, (0, 2, 1))


# speedup vs baseline: 1.1569x; 1.0019x over previous
import numpy as np
import jax
import jax.numpy as jnp
from jax.experimental import pallas as pl
from jax.experimental.pallas import tpu as pltpu

_PI = float(np.pi)
_SCALE = float(2.0 / (np.cos(0.0) - np.cos(2.0 * np.pi / 8.0)))
_LOG2E = float(np.log2(np.e))


def _poly_coeffs():
    r = np.linspace(-np.pi, np.pi, 40001)
    q = r * r
    V = np.stack([q**i for i in range(6)], axis=1)
    cc = np.linalg.lstsq(V, np.cos(r), rcond=None)[0]
    sc = np.linalg.lstsq(V * r[:, None], np.sin(r), rcond=None)[0]
    return [float(x) for x in cc], [float(x) for x in sc]


_CC, _SC = _poly_coeffs()


def _expand_matrix(nf, nb):
    F = nf * nb
    centers = np.arange(nb) * (2.0 * np.pi / nb)
    A = np.zeros((4 * F, 6 * nf), dtype=np.float64)
    for jj in range(2):
        for k in range(nb):
            for i in range(nf):
                A[jj * F + k * nf + i, jj * nf + i] = -np.cos(centers[k]) * _SCALE
                A[jj * F + k * nf + i, 2 * nf + jj * nf + i] = -np.sin(centers[k]) * _SCALE
                A[2 * F + jj * F + k * nf + i, 4 * nf + jj * nf + i] = 1.0
    return A.astype(np.float32)


def _body(f_ref, m_ref, a_ref, w4_ref, b4_ref, o_ref):
    bb, D = f_ref.shape[0], o_ref.shape[1]
    F2 = a_ref.shape[0] // 2
    for q0 in range(0, bb, 4):
        feats = []
        for p in (q0, q0 + 2):
            f2 = jnp.concatenate([f_ref[p], f_ref[p + 1]], axis=0)
            m2 = jnp.concatenate([m_ref[p], m_ref[p + 1]], axis=0)
            r = f2 - _PI
            q = r * r
            c = _CC[5]
            s = _SC[5]
            for i in range(4, -1, -1):
                c = c * q + _CC[i]
                s = s * q + _SC[i]
            s = s * r
            csm = jnp.concatenate([c, s, m2], axis=0)
            em = jnp.dot(a_ref[...], csm, preferred_element_type=jnp.float32)
            feats.append(jnp.exp(em[:F2] - _SCALE) * em[F2:])
        feat4 = jnp.concatenate(feats, axis=0)
        out4 = jnp.dot(w4_ref[...], feat4,
                       preferred_element_type=jnp.float32) + b4_ref[...]
        for u in range(4):
            o_ref[q0 + u] = out4[u * D:(u + 1) * D]


def kernel(field, W, b, mask_field):
    B, G, nf = field.shape
    nb = 8
    D = W.shape[1]
    F = nf * nb

    fT = jnp.transpose(field, (0, 2, 1))
    mT = jnp.transpose(mask_field, (0, 2, 1))

    A = _expand_matrix(nf, nb)
    perm = np.asarray([i * nb + k for k in range(nb) for i in range(nf)])
    WT = W.astype(jnp.float32)[perm].T
    W4 = jnp.kron(jnp.eye(4, dtype=jnp.float32), WT)
    b4 = jnp.tile(b.astype(jnp.float32).reshape(D, 1), (4, 1))

    bb = 64 if B % 64 == 0 else 4
    grid = (B // bb,)
    in3 = lambda c2: pl.BlockSpec((bb, c2, G), lambda i: (i, 0, 0))
    const_spec = lambda shape: pl.BlockSpec(shape, lambda i: (0, 0))

    flops = 2 * B * G * (2 * F * 3 * nf + D * F)
    transcendentals = B * G * F
    bytes_accessed = B * G * (2 * nf + D) * 4
    cost = pl.CostEstimate(flops=flops, transcendentals=transcendentals,
                           bytes_accessed=bytes_accessed)

    out = pl.pallas_call(
        _body,
        out_shape=jax.ShapeDtypeStruct((B, D, G), jnp.float32),
        grid_spec=pltpu.PrefetchScalarGridSpec(
            num_scalar_prefetch=0,
            grid=grid,
            in_specs=[in3(nf), in3(nf),
                      const_spec((4 * F, 6 * nf)),
                      const_spec((4 * D, 4 * F)), const_spec((4 * D, 1))],
            out_specs=pl.BlockSpec((bb, D, G), lambda i: (i, 0, 0)),
        ),
        compiler_params=pltpu.CompilerParams(
            dimension_semantics=("parallel",)),
        cost_estimate=cost,
    )(fT, mT, jnp.asarray(A), W4, b4)

    return jnp.transpose(out, (0, 2, 1))


# speedup vs baseline: 1.1571x; 1.0001x over previous
import numpy as np
import jax
import jax.numpy as jnp
from jax.experimental import pallas as pl
from jax.experimental.pallas import tpu as pltpu

_PI = float(np.pi)
_SCALE = float(2.0 / (np.cos(0.0) - np.cos(2.0 * np.pi / 8.0)))
_LOG2E = float(np.log2(np.e))


def _poly_coeffs():
    r = np.linspace(-np.pi, np.pi, 40001)
    q = r * r
    V = np.stack([q**i for i in range(6)], axis=1)
    cc = np.linalg.lstsq(V, np.cos(r), rcond=None)[0]
    sc = np.linalg.lstsq(V * r[:, None], np.sin(r), rcond=None)[0]
    return [float(x) for x in cc], [float(x) for x in sc]


_CC, _SC = _poly_coeffs()


def _expand_matrix(nf, nb):
    F = nf * nb
    centers = np.arange(nb) * (2.0 * np.pi / nb)
    A = np.zeros((4 * F, 6 * nf), dtype=np.float64)
    for jj in range(2):
        for k in range(nb):
            for i in range(nf):
                A[jj * F + k * nf + i, jj * nf + i] = -np.cos(centers[k])
                A[jj * F + k * nf + i, 2 * nf + jj * nf + i] = -np.sin(centers[k])
                A[2 * F + jj * F + k * nf + i, 4 * nf + jj * nf + i] = 1.0
    return A.astype(np.float32)


def _body(f_ref, m_ref, a_ref, w4_ref, b4_ref, o_ref):
    bb, D = f_ref.shape[0], o_ref.shape[1]
    F2 = a_ref.shape[0] // 2
    for q0 in range(0, bb, 4):
        feats = []
        for p in (q0, q0 + 2):
            f2 = jnp.concatenate([f_ref[p], f_ref[p + 1]], axis=0)
            m2 = jnp.concatenate([m_ref[p], m_ref[p + 1]], axis=0)
            r = f2 - _PI
            q = r * r
            c = _CC[5]
            s = _SC[5]
            for i in range(4, -1, -1):
                c = c * q + _CC[i]
                s = s * q + _SC[i]
            s = s * r
            csm = jnp.concatenate([c, s, m2], axis=0)
            em = jnp.dot(a_ref[...], csm, preferred_element_type=jnp.float32)
            feats.append(jnp.exp((em[:F2] - 1.0) * _SCALE) * em[F2:])
        feat4 = jnp.concatenate(feats, axis=0)
        out4 = jnp.dot(w4_ref[...], feat4,
                       preferred_element_type=jnp.float32) + b4_ref[...]
        for u in range(4):
            o_ref[q0 + u] = out4[u * D:(u + 1) * D]


def kernel(field, W, b, mask_field):
    B, G, nf = field.shape
    nb = 8
    D = W.shape[1]
    F = nf * nb

    fT = jnp.transpose(field, (0, 2, 1))
    mT = jnp.transpose(mask_field, (0, 2, 1))

    A = _expand_matrix(nf, nb)
    perm = np.asarray([i * nb + k for k in range(nb) for i in range(nf)])
    WT = W.astype(jnp.float32)[perm].T
    W4 = jnp.kron(jnp.eye(4, dtype=jnp.float32), WT)
    b4 = jnp.tile(b.astype(jnp.float32).reshape(D, 1), (4, 1))

    bb = 64 if B % 64 == 0 else 4
    grid = (B // bb,)
    in3 = lambda c2: pl.BlockSpec((bb, c2, G), lambda i: (i, 0, 0))
    const_spec = lambda shape: pl.BlockSpec(shape, lambda i: (0, 0))

    flops = 2 * B * G * (2 * F * 3 * nf + D * F)
    transcendentals = B * G * F
    bytes_accessed = B * G * (2 * nf + D) * 4
    cost = pl.CostEstimate(flops=flops, transcendentals=transcendentals,
                           bytes_accessed=bytes_accessed)

    out = pl.pallas_call(
        _body,
        out_shape=jax.ShapeDtypeStruct((B, D, G), jnp.float32),
        grid_spec=pltpu.PrefetchScalarGridSpec(
            num_scalar_prefetch=0,
            grid=grid,
            in_specs=[in3(nf), in3(nf),
                      const_spec((4 * F, 6 * nf)),
                      const_spec((4 * D, 4 * F)), const_spec((4 * D, 1))],
            out_specs=pl.BlockSpec((bb, D, G), lambda i: (i, 0, 0)),
        ),
        compiler_params=pltpu.CompilerParams(
            dimension_semantics=("parallel",)),
        cost_estimate=cost,
    )(fT, mT, jnp.asarray(A), W4, b4)

    return jnp.transpose(out, (0, 2, 1))
